# baseline (speedup 1.0000x reference)
import functools

import jax
import jax.numpy as jnp
from jax import lax
from jax.experimental import pallas as pl
from jax.experimental.pallas import tpu as pltpu
from jax.experimental.pallas import tpu_sc as plsc

B = 1024
N = 20000
N_DN = 1300
N_MBON = 96
NC = 2
NS = 16
NW = NC * NS
DN_PAD = 1312
IDX_PAD = DN_PAD + N_MBON
BIG = 48
SMALL = 32
N_BIG = 24
BIG_ROWS = N_BIG * BIG
DN_TAIL_W = 28
MB_W0 = 29


def _sc_body2(vt_hbm, cidx_hbm, dnt_hbm, mbt_hbm, cidx_v, gbuf, sg):
    wid = lax.axis_index("s") * NC + lax.axis_index("c")

    pltpu.sync_copy(cidx_hbm, cidx_v)

    is_big = wid < N_BIG
    is_dn_small = (wid >= N_BIG) & (wid < DN_TAIL_W)
    is_dn_tail = wid == DN_TAIL_W
    is_mb = wid >= MB_W0

    def gcopy(nrows, off):
        return pltpu.make_async_copy(
            vt_hbm.at[cidx_v.at[pl.ds(off, nrows)]],
            gbuf.at[pl.ds(0, nrows)], sg)

    small_off = BIG_ROWS + (wid - N_BIG) * SMALL

    @pl.when(is_big)
    def _():
        gcopy(BIG, wid * BIG).start()

    @pl.when(~is_big)
    def _():
        gcopy(SMALL, small_off).start()

    @pl.when(is_big)
    def _():
        gcopy(BIG, wid * BIG).wait()
        pltpu.sync_copy(gbuf.at[pl.ds(0, BIG)],
                        dnt_hbm.at[pl.ds(wid * BIG, BIG)])

    @pl.when(is_dn_small)
    def _():
        gcopy(SMALL, small_off).wait()
        pltpu.sync_copy(gbuf.at[pl.ds(0, SMALL)],
                        dnt_hbm.at[pl.ds(small_off, SMALL)])

    @pl.when(is_dn_tail)
    def _():
        gcopy(SMALL, small_off).wait()
        pltpu.sync_copy(gbuf.at[pl.ds(0, 16)],
                        dnt_hbm.at[pl.ds(small_off, 16)])
        pltpu.sync_copy(gbuf.at[pl.ds(16, 4)],
                        dnt_hbm.at[pl.ds(small_off + 16, 4)])

    @pl.when(is_mb)
    def _():
        gcopy(SMALL, small_off).wait()
        pltpu.sync_copy(gbuf.at[pl.ds(0, SMALL)],
                        mbt_hbm.at[pl.ds((wid - MB_W0) * SMALL, SMALL)])


@jax.jit
def kernel(v, dn_id, mbon_id):
    vt = jnp.swapaxes(v, 0, 1)
    cidx = jnp.concatenate(
        [dn_id.astype(jnp.int32),
         jnp.zeros((DN_PAD - N_DN,), jnp.int32),
         mbon_id.astype(jnp.int32)])

    mesh = plsc.VectorSubcoreMesh(core_axis_name="c", subcore_axis_name="s")
    run = pl.kernel(
        _sc_body2,
        mesh=mesh,
        compiler_params=pltpu.CompilerParams(needs_layout_passes=False,
                                             use_tc_tiling_on_sc=True),
        out_type=(jax.ShapeDtypeStruct((N_DN, B), jnp.float32),
                  jax.ShapeDtypeStruct((N_MBON, B), jnp.float32)),
        scratch_types=[
            pltpu.VMEM((IDX_PAD,), jnp.int32),
            pltpu.VMEM((BIG, B), jnp.float32),
            pltpu.SemaphoreType.DMA,
        ],
    )
    dnt, mbt = run(vt, cidx)
    return jnp.swapaxes(dnt, 0, 1), jnp.swapaxes(mbt, 0, 1)


# speedup vs baseline: 1.0105x; 1.0105x over previous
import functools

import jax
import jax.numpy as jnp
from jax import lax
from jax.experimental import pallas as pl
from jax.experimental.pallas import tpu as pltpu
from jax.experimental.pallas import tpu_sc as plsc

B = 1024
N = 20000
N_DN = 1300
N_MBON = 96
NC = 2
NS = 16
NW = NC * NS
BLK = 16
DN_BLKS = (N_DN + BLK - 1) // BLK
DN_TAIL = N_DN - (DN_BLKS - 1) * BLK
MB_BLKS = N_MBON // BLK
TOT_BLKS = DN_BLKS + MB_BLKS
IDX_PAD = TOT_BLKS * BLK
MAX_BLKS_PER_W = (TOT_BLKS + NW - 1) // NW
NBUF = 3


def _sc_body(vt_hbm, dn_idx_hbm, mbon_idx_hbm, dnt_hbm, mbt_hbm,
             cidx_v, g0, g1, g2, sg0, sg1, sg2, so0, so1, so2):
    wid = lax.axis_index("s") * NC + lax.axis_index("c")
    gb = (g0, g1, g2)
    sg = (sg0, sg1, sg2)
    so = (so0, so1, so2)

    c1 = pltpu.make_async_copy(dn_idx_hbm, cidx_v.at[pl.ds(0, N_DN)], sg0)
    c2 = pltpu.make_async_copy(mbon_idx_hbm,
                               cidx_v.at[pl.ds(DN_BLKS * BLK, N_MBON)], sg1)
    c1.start()
    c2.start()
    c1.wait()
    c2.wait()
    pad_at = (DN_BLKS - 1) * BLK
    chunk = cidx_v[pl.ds(pad_at, BLK)]
    cidx_v[pl.ds(pad_at, BLK)] = jnp.where(
        lax.iota(jnp.int32, BLK) < DN_TAIL, chunk, 0)

    def gather_copy(k, blk):
        return pltpu.make_async_copy(
            vt_hbm.at[cidx_v.at[pl.ds(blk * BLK, BLK)]],
            gb[k % NBUF], sg[k % NBUF])

    def full_dn_copy(k, blk):
        return pltpu.make_async_copy(
            gb[k % NBUF], dnt_hbm.at[pl.ds(blk * BLK, BLK)], so[k % NBUF])

    def part_dn_copy(k):
        return pltpu.make_async_copy(
            gb[k % NBUF].at[pl.ds(0, DN_TAIL)],
            dnt_hbm.at[pl.ds((DN_BLKS - 1) * BLK, DN_TAIL)], so[k % NBUF])

    def mb_copy(k, blk):
        return pltpu.make_async_copy(
            gb[k % NBUF], mbt_hbm.at[pl.ds((blk - DN_BLKS) * BLK, BLK)],
            so[k % NBUF])

    def issue_gather(k):
        blk = wid + NW * k

        @pl.when(blk < TOT_BLKS)
        def _():
            gather_copy(k, blk).start()

    def wait_gather(k):
        blk = wid + NW * k

        @pl.when(blk < TOT_BLKS)
        def _():
            gather_copy(k, blk).wait()

    def each_out(k, fn):
        blk = wid + NW * k

        @pl.when(blk < DN_BLKS - 1)
        def _():
            fn(full_dn_copy(k, blk))

        @pl.when(blk == DN_BLKS - 1)
        def _():
            fn(part_dn_copy(k))

        @pl.when((blk >= DN_BLKS) & (blk < TOT_BLKS))
        def _():
            fn(mb_copy(k, blk))

    for k in range(min(NBUF, MAX_BLKS_PER_W)):
        issue_gather(k)
    for k in range(MAX_BLKS_PER_W):
        if k >= NBUF:
            each_out(k - NBUF, lambda c: c.wait())
            issue_gather(k)
        wait_gather(k)
        each_out(k, lambda c: c.start())
    for k in range(max(0, MAX_BLKS_PER_W - NBUF), MAX_BLKS_PER_W):
        each_out(k, lambda c: c.wait())


@jax.jit
def kernel(v, dn_id, mbon_id):
    vt = jnp.swapaxes(v, 0, 1)

    mesh = plsc.VectorSubcoreMesh(core_axis_name="c", subcore_axis_name="s")
    run = pl.kernel(
        _sc_body,
        mesh=mesh,
        compiler_params=pltpu.CompilerParams(needs_layout_passes=False,
                                             use_tc_tiling_on_sc=True),
        out_type=(jax.ShapeDtypeStruct((N_DN, B), jnp.float32),
                  jax.ShapeDtypeStruct((N_MBON, B), jnp.float32)),
        scratch_types=(
            [pltpu.VMEM((IDX_PAD,), jnp.int32)]
            + [pltpu.VMEM((BLK, B), jnp.float32) for _ in range(NBUF)]
            + [pltpu.SemaphoreType.DMA for _ in range(2 * NBUF)]
        ),
    )
    dnt, mbt = run(vt, dn_id.astype(jnp.int32), mbon_id.astype(jnp.int32))
    return jnp.swapaxes(dnt, 0, 1), jnp.swapaxes(mbt, 0, 1)


# speedup vs baseline: 1.0116x; 1.0012x over previous
import jax
import jax.numpy as jnp
from jax import lax
from jax.experimental import pallas as pl
from jax.experimental.pallas import tpu as pltpu
from jax.experimental.pallas import tpu_sc as plsc

B = 1024
N = 20000
N_DN = 1300
N_MBON = 96
NC = 2
NS = 16
NW = NC * NS
BLK = 16
DN_BLKS = (N_DN + BLK - 1) // BLK
DN_TAIL = N_DN - (DN_BLKS - 1) * BLK
MB_BLKS = N_MBON // BLK
TOT_BLKS = DN_BLKS + MB_BLKS
IDX_PAD = TOT_BLKS * BLK
MAX_BLKS_PER_W = (TOT_BLKS + NW - 1) // NW
NBUF = 3


def _sc_body(vt_hbm, dn_idx_hbm, mbon_idx_hbm, dnt_hbm, mbt_hbm,
             cidx_v, g0, g1, g2, sg0, sg1, sg2, so0, so1, so2):
    wid = lax.axis_index("s") * NC + lax.axis_index("c")
    gb = (g0, g1, g2)
    sg = (sg0, sg1, sg2)
    so = (so0, so1, so2)

    c1 = pltpu.make_async_copy(dn_idx_hbm, cidx_v.at[pl.ds(0, N_DN)], sg0)
    c2 = pltpu.make_async_copy(mbon_idx_hbm,
                               cidx_v.at[pl.ds(DN_BLKS * BLK, N_MBON)], sg1)
    c1.start()
    c2.start()
    c1.wait()
    c2.wait()
    pad_at = (DN_BLKS - 1) * BLK
    chunk = cidx_v[pl.ds(pad_at, BLK)]
    cidx_v[pl.ds(pad_at, BLK)] = jnp.where(
        lax.iota(jnp.int32, BLK) < DN_TAIL, chunk, 0)

    def gather_copy(k, blk):
        return pltpu.make_async_copy(
            vt_hbm.at[cidx_v.at[pl.ds(blk * BLK, BLK)]],
            gb[k % NBUF], sg[k % NBUF])

    def full_dn_copy(k, blk):
        return pltpu.make_async_copy(
            gb[k % NBUF], dnt_hbm.at[pl.ds(blk * BLK, BLK)], so[k % NBUF])

    def part_dn_copy(k):
        return pltpu.make_async_copy(
            gb[k % NBUF].at[pl.ds(0, DN_TAIL)],
            dnt_hbm.at[pl.ds((DN_BLKS - 1) * BLK, DN_TAIL)], so[k % NBUF])

    def mb_copy(k, blk):
        return pltpu.make_async_copy(
            gb[k % NBUF], mbt_hbm.at[pl.ds((blk - DN_BLKS) * BLK, BLK)],
            so[k % NBUF])

    def issue_gather(k):
        blk = wid + NW * k

        @pl.when(blk < TOT_BLKS)
        def _():
            gather_copy(k, blk).start()

    def wait_gather(k):
        blk = wid + NW * k

        @pl.when(blk < TOT_BLKS)
        def _():
            gather_copy(k, blk).wait()

    def each_out(k, fn):
        blk = wid + NW * k

        @pl.when(blk < DN_BLKS - 1)
        def _():
            fn(full_dn_copy(k, blk))

        @pl.when(blk == DN_BLKS - 1)
        def _():
            fn(part_dn_copy(k))

        @pl.when((blk >= DN_BLKS) & (blk < TOT_BLKS))
        def _():
            fn(mb_copy(k, blk))

    for k in range(min(NBUF, MAX_BLKS_PER_W)):
        issue_gather(k)
    for k in range(MAX_BLKS_PER_W):
        if k >= NBUF:
            each_out(k - NBUF, lambda c: c.wait())
            issue_gather(k)
        wait_gather(k)
        each_out(k, lambda c: c.start())
    for k in range(max(0, MAX_BLKS_PER_W - NBUF), MAX_BLKS_PER_W):
        each_out(k, lambda c: c.wait())


@jax.jit
def kernel(v, dn_id, mbon_id):
    vt = jnp.swapaxes(v, 0, 1)

    mesh = plsc.VectorSubcoreMesh(core_axis_name="c", subcore_axis_name="s")
    run = pl.kernel(
        _sc_body,
        mesh=mesh,
        compiler_params=pltpu.CompilerParams(needs_layout_passes=False,
                                             use_tc_tiling_on_sc=True),
        out_type=(jax.ShapeDtypeStruct((N_DN, B), jnp.float32),
                  jax.ShapeDtypeStruct((N_MBON, B), jnp.float32)),
        scratch_types=(
            [pltpu.VMEM((IDX_PAD,), jnp.int32)]
            + [pltpu.VMEM((BLK, B), jnp.float32) for _ in range(NBUF)]
            + [pltpu.SemaphoreType.DMA for _ in range(2 * NBUF)]
        ),
    )
    dnt, mbt = run(vt, dn_id.astype(jnp.int32), mbon_id.astype(jnp.int32))
    return jnp.swapaxes(dnt, 0, 1), jnp.swapaxes(mbt, 0, 1)
